# baseline (speedup 1.0000x reference)
import jax
import jax.numpy as jnp
from jax import lax
from jax.experimental import pallas as pl
from jax.experimental.pallas import tpu as pltpu

_CHUNK = 128
_NBITS = 4
_NEXP = 16
_KTILE = 512


def _route_kernel(x_ref, hp_ref, cnt_ref, tab_ref, flag_ref, xbf_ref):
    nc = x_ref.shape[0]
    na = 2 * nc
    xr = x_ref[...]
    emb = jnp.mean(xr, axis=1)
    proj = jnp.dot(emb, hp_ref[...],
                   preferred_element_type=jnp.float32)
    bits = (proj > 0).astype(jnp.int32)
    col = lax.broadcasted_iota(jnp.int32, (nc, _NBITS), 1)
    powers = jnp.left_shift(jnp.ones((nc, _NBITS), jnp.int32), col)
    e1 = jnp.sum(bits * powers, axis=1, keepdims=True)
    ap = jnp.abs(proj)
    mn = jnp.min(ap, axis=1, keepdims=True)
    cand = jnp.where(ap == mn, col, _NBITS)
    weak = jnp.min(cand, axis=1, keepdims=True)
    flip = jnp.left_shift(jnp.ones_like(weak), weak)
    e2 = jnp.bitwise_xor(e1, flip)

    ra = lax.broadcasted_iota(jnp.int32, (nc, nc), 0)
    rb = lax.broadcasted_iota(jnp.int32, (nc, nc), 1)
    onesrow = jnp.ones((1, nc), jnp.float32)

    def _to_row(colvec):
        dm = jnp.where(ra == rb, jnp.broadcast_to(
            colvec.astype(jnp.float32), (nc, nc)), 0.0)
        return jnp.dot(onesrow, dm, preferred_element_type=jnp.float32)

    e1r = _to_row(e1)
    e2r = _to_row(e2)
    eminr = _to_row(jnp.minimum(e1, e2))
    eflat = jnp.concatenate([e1r, e2r], axis=1)
    eminf = jnp.concatenate([eminr, eminr], axis=1)
    ck = (lax.broadcasted_iota(jnp.int32, (1, na), 1) % nc).astype(jnp.float32)

    erow = lax.broadcasted_iota(jnp.int32, (_NEXP, na), 0).astype(jnp.float32)
    match = (jnp.broadcast_to(eflat, (_NEXP, na)) == erow).astype(jnp.float32)
    ka = lax.broadcasted_iota(jnp.int32, (na, na), 0)
    kb = lax.broadcasted_iota(jnp.int32, (na, na), 1)
    lower = (ka < kb).astype(jnp.float32)
    rank = jnp.dot(match, lower, preferred_element_type=jnp.float32)
    cnt_ref[...] = jnp.sum(match, axis=1, keepdims=True).astype(jnp.int32)

    isfirst = (eflat == eminf).astype(jnp.float32)
    ja = lax.broadcasted_iota(jnp.int32, (1, na), 1).astype(jnp.float32)
    tab_rows, flag_rows = [], []
    for e in range(_NEXP):
        onehot = (rank[e:e + 1, :].reshape(na, 1) ==
                  jnp.broadcast_to(ja, (na, na))).astype(jnp.float32)
        onehot = onehot * match[e:e + 1, :].reshape(na, 1)
        tab_rows.append(jnp.dot(ck, onehot,
                                preferred_element_type=jnp.float32))
        flag_rows.append(jnp.dot(ck * 0 + isfirst, onehot,
                                 preferred_element_type=jnp.float32))
    tab_ref[...] = jnp.concatenate(tab_rows, axis=0).astype(jnp.int32)
    flag_ref[...] = jnp.concatenate(flag_rows, axis=0).astype(jnp.int32)

    xbf_ref[...] = xr.astype(jnp.bfloat16).reshape(nc * _CHUNK, xr.shape[2])


def kernel(x, hyperplanes, expert_weights):
    bsz, seq, d = x.shape
    nc = (bsz * seq) // _CHUNK
    na = 2 * nc
    rows = nc * _CHUNK
    ne = expert_weights.shape[0]
    x3 = x.reshape(nc, _CHUNK, d)

    cnt, tab, flag, xbf = pl.pallas_call(
        _route_kernel,
        out_shape=[
            jax.ShapeDtypeStruct((ne, 1), jnp.int32),
            jax.ShapeDtypeStruct((ne, na), jnp.int32),
            jax.ShapeDtypeStruct((ne, na), jnp.int32),
            jax.ShapeDtypeStruct((rows, d), jnp.bfloat16),
        ],
    )(x3, hyperplanes)

    nkt = d // _KTILE
    total = ne * nkt

    def _wcopy(w_hbm, wf_ref, sem_w, t):
        et = t // nkt
        ktt = t % nkt
        return pltpu.make_async_copy(
            w_hbm.at[et, pl.ds(ktt * _KTILE, _KTILE), :],
            wf_ref.at[t % 2], sem_w.at[t % 2])

    def _moe_kernel(cnt_ref, tab_ref, flag_ref, x_hbm, w_hbm, o_ref,
                    xs_ref, wf_ref, wb_ref, sem_x, sem_w):
        e = pl.program_id(0)
        kt = pl.program_id(1)
        s = e * nkt + kt

        @pl.when(s == 0)
        def _prologue():
            cpx = pltpu.make_async_copy(x_hbm, xs_ref, sem_x)
            cpx.start()
            _wcopy(w_hbm, wf_ref, sem_w, 0).start()
            cpx.wait()

        @pl.when(s + 1 < total)
        def _issue_next():
            _wcopy(w_hbm, wf_ref, sem_w, s + 1).start()

        _wcopy(w_hbm, wf_ref, sem_w, s).wait()
        wb_ref[...] = (wf_ref[s % 2] * 0.5).astype(jnp.bfloat16)

        def _body(j, _):
            c = tab_ref[e, j]
            xs = xs_ref[pl.ds(c * _CHUNK, _CHUNK), pl.ds(kt * _KTILE, _KTILE)]
            contrib = jnp.dot(xs, wb_ref[...],
                              preferred_element_type=jnp.float32)
            prev = o_ref[pl.ds(c * _CHUNK, _CHUNK), :]
            base = jnp.where((flag_ref[e, j] != 0) & (kt == 0), 0.0, prev)
            o_ref[pl.ds(c * _CHUNK, _CHUNK), :] = base + contrib
            return 0

        lax.fori_loop(0, cnt_ref[e, 0], _body, 0)

    grid_spec = pltpu.PrefetchScalarGridSpec(
        num_scalar_prefetch=3,
        grid=(ne, nkt),
        in_specs=[
            pl.BlockSpec(memory_space=pl.ANY),
            pl.BlockSpec(memory_space=pl.ANY),
        ],
        out_specs=pl.BlockSpec((rows, d), lambda e, kt, *_: (0, 0)),
        scratch_shapes=[
            pltpu.VMEM((rows, d), jnp.bfloat16),
            pltpu.VMEM((2, _KTILE, d), jnp.float32),
            pltpu.VMEM((_KTILE, d), jnp.bfloat16),
            pltpu.SemaphoreType.DMA,
            pltpu.SemaphoreType.DMA((2,)),
        ],
    )
    out = pl.pallas_call(
        _moe_kernel,
        grid_spec=grid_spec,
        out_shape=jax.ShapeDtypeStruct((rows, d), jnp.float32),
    )(cnt, tab, flag, xbf, expert_weights)
    return out.reshape(bsz, seq, d)


# speedup vs baseline: 1.0049x; 1.0049x over previous
import jax
import jax.numpy as jnp
from jax import lax
from jax.experimental import pallas as pl
from jax.experimental.pallas import tpu as pltpu

_CHUNK = 128
_NBITS = 4
_NEXP = 16
_KTILE = 512


def _route_kernel(x_ref, hp_ref, cnt_ref, tab_ref, flag_ref, xbf_ref):
    nc = x_ref.shape[0]
    na = 2 * nc
    xr = x_ref[...]
    emb = jnp.mean(xr, axis=1)
    proj = jnp.dot(emb, hp_ref[...],
                   preferred_element_type=jnp.float32)
    bits = (proj > 0).astype(jnp.int32)
    col = lax.broadcasted_iota(jnp.int32, (nc, _NBITS), 1)
    powers = jnp.left_shift(jnp.ones((nc, _NBITS), jnp.int32), col)
    e1 = jnp.sum(bits * powers, axis=1, keepdims=True)
    ap = jnp.abs(proj)
    mn = jnp.min(ap, axis=1, keepdims=True)
    cand = jnp.where(ap == mn, col, _NBITS)
    weak = jnp.min(cand, axis=1, keepdims=True)
    flip = jnp.left_shift(jnp.ones_like(weak), weak)
    e2 = jnp.bitwise_xor(e1, flip)

    ra = lax.broadcasted_iota(jnp.int32, (nc, nc), 0)
    rb = lax.broadcasted_iota(jnp.int32, (nc, nc), 1)
    onesrow = jnp.ones((1, nc), jnp.float32)

    def _to_row(colvec):
        dm = jnp.where(ra == rb, jnp.broadcast_to(
            colvec.astype(jnp.float32), (nc, nc)), 0.0)
        return jnp.dot(onesrow, dm, preferred_element_type=jnp.float32)

    e1r = _to_row(e1)
    e2r = _to_row(e2)
    eminr = _to_row(jnp.minimum(e1, e2))
    eflat = jnp.concatenate([e1r, e2r], axis=1)
    eminf = jnp.concatenate([eminr, eminr], axis=1)
    ck = (lax.broadcasted_iota(jnp.int32, (1, na), 1) % nc).astype(jnp.float32)

    erow = lax.broadcasted_iota(jnp.int32, (_NEXP, na), 0).astype(jnp.float32)
    match = (jnp.broadcast_to(eflat, (_NEXP, na)) == erow).astype(jnp.float32)
    ka = lax.broadcasted_iota(jnp.int32, (na, na), 0)
    kb = lax.broadcasted_iota(jnp.int32, (na, na), 1)
    lower = (ka < kb).astype(jnp.float32)
    rank = jnp.dot(match, lower, preferred_element_type=jnp.float32)
    cnt_ref[...] = jnp.sum(match, axis=1, keepdims=True).astype(jnp.int32)

    isfirst = (eflat == eminf).astype(jnp.float32)
    ja = lax.broadcasted_iota(jnp.int32, (1, na), 1).astype(jnp.float32)
    tab_rows, flag_rows = [], []
    for e in range(_NEXP):
        onehot = (rank[e:e + 1, :].reshape(na, 1) ==
                  jnp.broadcast_to(ja, (na, na))).astype(jnp.float32)
        onehot = onehot * match[e:e + 1, :].reshape(na, 1)
        tab_rows.append(jnp.dot(ck, onehot,
                                preferred_element_type=jnp.float32))
        flag_rows.append(jnp.dot(ck * 0 + isfirst, onehot,
                                 preferred_element_type=jnp.float32))
    tab_ref[...] = jnp.concatenate(tab_rows, axis=0).astype(jnp.int32)
    flag_ref[...] = jnp.concatenate(flag_rows, axis=0).astype(jnp.int32)

    xbf_ref[...] = xr.astype(jnp.bfloat16).reshape(nc * _CHUNK, xr.shape[2])


def kernel(x, hyperplanes, expert_weights):
    bsz, seq, d = x.shape
    nc = (bsz * seq) // _CHUNK
    na = 2 * nc
    rows = nc * _CHUNK
    ne = expert_weights.shape[0]
    x3 = x.reshape(nc, _CHUNK, d)

    cnt, tab, flag, xbf = pl.pallas_call(
        _route_kernel,
        out_shape=[
            jax.ShapeDtypeStruct((ne, 1), jnp.int32),
            jax.ShapeDtypeStruct((ne, na), jnp.int32),
            jax.ShapeDtypeStruct((ne, na), jnp.int32),
            jax.ShapeDtypeStruct((rows, d), jnp.bfloat16),
        ],
    )(x3, hyperplanes)

    nkt = d // _KTILE
    total = ne * nkt

    def _wcopy(w_hbm, wf_ref, sem_w, t):
        et = t // nkt
        ktt = t % nkt
        return pltpu.make_async_copy(
            w_hbm.at[et, pl.ds(ktt * _KTILE, _KTILE), :],
            wf_ref.at[t % 2], sem_w.at[t % 2])

    def _moe_kernel(cnt_ref, tab_ref, flag_ref, x_hbm, w_hbm, o_ref,
                    xs_ref, wf_ref, wb_ref, sem_x, sem_w):
        e = pl.program_id(0)
        kt = pl.program_id(1)
        s = e * nkt + kt

        @pl.when(s == 0)
        def _prologue():
            cpx = pltpu.make_async_copy(x_hbm, xs_ref, sem_x)
            cpx.start()
            _wcopy(w_hbm, wf_ref, sem_w, 0).start()
            cpx.wait()

        @pl.when(s + 1 < total)
        def _issue_next():
            _wcopy(w_hbm, wf_ref, sem_w, s + 1).start()

        _wcopy(w_hbm, wf_ref, sem_w, s).wait()
        wb_ref[...] = (wf_ref[s % 2] * 0.5).astype(jnp.bfloat16)

        cntv = cnt_ref[e, 0]

        def _dot1(j):
            c = tab_ref[e, j]
            xs = xs_ref[pl.ds(c * _CHUNK, _CHUNK), pl.ds(kt * _KTILE, _KTILE)]
            contrib = jnp.dot(xs, wb_ref[...],
                              preferred_element_type=jnp.float32)
            prev = o_ref[pl.ds(c * _CHUNK, _CHUNK), :]
            base = jnp.where((flag_ref[e, j] != 0) & (kt == 0), 0.0, prev)
            o_ref[pl.ds(c * _CHUNK, _CHUNK), :] = base + contrib

        def _body(i, _):
            j = i * 4
            _dot1(j)
            for off in (1, 2, 3):
                @pl.when(j + off < cntv)
                def _tail(j=j, off=off):
                    _dot1(j + off)
            return 0

        lax.fori_loop(0, (cntv + 3) // 4, _body, 0)

    grid_spec = pltpu.PrefetchScalarGridSpec(
        num_scalar_prefetch=3,
        grid=(ne, nkt),
        in_specs=[
            pl.BlockSpec(memory_space=pl.ANY),
            pl.BlockSpec(memory_space=pl.ANY),
        ],
        out_specs=pl.BlockSpec((rows, d), lambda e, kt, *_: (0, 0)),
        scratch_shapes=[
            pltpu.VMEM((rows, d), jnp.bfloat16),
            pltpu.VMEM((2, _KTILE, d), jnp.float32),
            pltpu.VMEM((_KTILE, d), jnp.bfloat16),
            pltpu.SemaphoreType.DMA,
            pltpu.SemaphoreType.DMA((2,)),
        ],
    )
    out = pl.pallas_call(
        _moe_kernel,
        grid_spec=grid_spec,
        out_shape=jax.ShapeDtypeStruct((rows, d), jnp.float32),
    )(cnt, tab, flag, xbf, expert_weights)
    return out.reshape(bsz, seq, d)
